# baseline (speedup 1.0000x reference)
import functools

import jax
import jax.numpy as jnp
from jax import lax
from jax.experimental import pallas as pl
from jax.experimental.pallas import tpu as pltpu
from jax.experimental.pallas import tpu_sc as plsc

_F = 128
_OUTW = 2 * _F
_NC = 2
_NS = 16
_NW = _NC * _NS
_LANES = 16
_CHUNK = 64


@functools.lru_cache(maxsize=None)
def _make_kernel(P: int):
    assert P % _NW == 0
    ppw = P // _NW
    assert ppw % _CHUNK == 0
    nch = ppw // _CHUNK

    mesh = plsc.VectorSubcoreMesh(
        core_axis_name="c", subcore_axis_name="s",
        num_cores=_NC, num_subcores=_NS)

    @functools.partial(
        pl.kernel,
        out_type=jax.ShapeDtypeStruct((P * _OUTW,), jnp.float32),
        mesh=mesh,
        scratch_types=[
            pltpu.VMEM((2 * ppw,), jnp.int32),
            pltpu.VMEM((ppw,), jnp.int32),
            pltpu.VMEM((ppw,), jnp.int32),
            pltpu.VMEM((_CHUNK, _F), jnp.float32),
            pltpu.VMEM((_CHUNK, _F), jnp.float32),
            pltpu.VMEM((_CHUNK * _OUTW,), jnp.float32),
            pltpu.SemaphoreType.DMA,
        ],
        compiler_params=pltpu.CompilerParams(needs_layout_passes=False),
    )
    def emb(x_hbm, col_hbm, row_hbm, out_hbm,
            xbuf, idx0, idx1, buf_a, buf_b, buf_c, sem):
        wid = lax.axis_index("s") * _NC + lax.axis_index("c")
        base = wid * ppw
        pltpu.sync_copy(x_hbm.at[pl.ds(2 * base, 2 * ppw)], xbuf)
        ev = 2 * lax.iota(jnp.int32, _LANES)

        def split_idx(k, c0):
            b2 = 2 * _LANES * k
            idx0[pl.ds(k * _LANES, _LANES)] = plsc.load_gather(xbuf, [b2 + ev])
            idx1[pl.ds(k * _LANES, _LANES)] = plsc.load_gather(xbuf, [b2 + 1 + ev])
            return c0

        lax.fori_loop(0, ppw // _LANES, split_idx, 0)

        def do_chunk(ci, carry):
            off = ci * _CHUNK
            ga = pltpu.async_copy(
                col_hbm.at[idx0.at[pl.ds(off, _CHUNK)]], buf_a, sem)
            gb = pltpu.async_copy(
                row_hbm.at[idx1.at[pl.ds(off, _CHUNK)]], buf_b, sem)
            ga.wait()
            gb.wait()

            def do_point(p, c2):
                pb = p * _OUTW
                for j in range(_F // _LANES):
                    va = buf_a[p, pl.ds(j * _LANES, _LANES)]
                    plsc.store_scatter(buf_c, [pb + 2 * j * _LANES + ev], va)
                    vb = buf_b[p, pl.ds(j * _LANES, _LANES)]
                    plsc.store_scatter(buf_c, [pb + 2 * j * _LANES + 1 + ev], vb)
                return c2

            lax.fori_loop(0, _CHUNK, do_point, 0)
            pltpu.sync_copy(
                buf_c, out_hbm.at[pl.ds((base + off) * _OUTW, _CHUNK * _OUTW)])
            return carry

        lax.fori_loop(0, nch, do_chunk, 0)

    return emb


def kernel(x, col_embed, row_embed):
    b, n, _ = x.shape
    p = b * n
    out = _make_kernel(p)(x.reshape(2 * p), col_embed, row_embed)
    return out.reshape(b, n, _F, 2)


# speedup vs baseline: 48.9448x; 48.9448x over previous
import functools

import jax
import jax.numpy as jnp
from jax import lax
from jax.experimental import pallas as pl
from jax.experimental.pallas import tpu as pltpu
from jax.experimental.pallas import tpu_sc as plsc

_F = 128
_NC = 2
_NS = 16
_NW = _NC * _NS
_LANES = 16
_CHUNK = 64


@functools.lru_cache(maxsize=None)
def _make_kernel(P: int):
    assert P % _NW == 0
    ppw = P // _NW
    assert ppw % _CHUNK == 0
    nch = ppw // _CHUNK
    assert nch % 2 == 0
    rows = 2 * _CHUNK

    mesh = plsc.VectorSubcoreMesh(
        core_axis_name="c", subcore_axis_name="s",
        num_cores=_NC, num_subcores=_NS)

    @functools.partial(
        pl.kernel,
        out_type=jax.ShapeDtypeStruct((2 * P, _F), jnp.float32),
        mesh=mesh,
        scratch_types=[
            pltpu.VMEM((2 * ppw,), jnp.int32),
            pltpu.VMEM((rows, _F), jnp.float32),
            pltpu.VMEM((rows, _F), jnp.float32),
            pltpu.SemaphoreType.DMA,
            pltpu.SemaphoreType.DMA,
            pltpu.SemaphoreType.DMA,
            pltpu.SemaphoreType.DMA,
        ],
        compiler_params=pltpu.CompilerParams(needs_layout_passes=False),
    )
    def emb(x_hbm, tab_hbm, out_hbm,
            idx, buf0, buf1, gs0, gs1, os0, os1):
        wid = lax.axis_index("s") * _NC + lax.axis_index("c")
        base = wid * ppw
        pltpu.sync_copy(x_hbm.at[pl.ds(2 * base, 2 * ppw)], idx)

        odd512 = 512 * (lax.iota(jnp.int32, _LANES) & 1)

        def fuse(k, c0):
            s = pl.ds(k * _LANES, _LANES)
            idx[s] = idx[s] + odd512
            return c0

        lax.fori_loop(0, 2 * ppw // _LANES, fuse, 0)

        def gather(ci, buf, sem):
            r0 = ci * rows
            pltpu.async_copy(tab_hbm.at[idx.at[pl.ds(r0, rows)]], buf, sem)

        def gather_wait(buf, sem):
            pltpu.make_async_copy(
                tab_hbm.at[idx.at[pl.ds(0, rows)]], buf, sem).wait()

        def writeback(ci, buf, sem):
            r0 = 2 * base + ci * rows
            pltpu.async_copy(buf, out_hbm.at[pl.ds(r0, rows)], sem)

        def writeback_wait(buf, sem):
            pltpu.make_async_copy(
                buf, out_hbm.at[pl.ds(2 * base, rows)], sem).wait()

        gather(0, buf0, gs0)

        def stage(i, c0):
            gather_wait(buf0, gs0)

            @pl.when(i > 0)
            def _():
                writeback_wait(buf1, os1)

            gather(2 * i + 1, buf1, gs1)
            writeback(2 * i, buf0, os0)
            gather_wait(buf1, gs1)
            writeback_wait(buf0, os0)

            @pl.when(i < nch // 2 - 1)
            def _():
                gather(2 * i + 2, buf0, gs0)

            writeback(2 * i + 1, buf1, os1)
            return c0

        lax.fori_loop(0, nch // 2, stage, 0)
        writeback_wait(buf1, os1)

    return emb


def kernel(x, col_embed, row_embed):
    b, n, _ = x.shape
    p = b * n
    tab = jnp.concatenate([col_embed, row_embed], axis=0)
    out = _make_kernel(p)(x.reshape(2 * p), tab)
    return out.reshape(b, n, 2, _F).swapaxes(2, 3)


# speedup vs baseline: 50.8998x; 1.0399x over previous
import functools

import jax
import jax.numpy as jnp
from jax import lax
from jax.experimental import pallas as pl
from jax.experimental.pallas import tpu as pltpu
from jax.experimental.pallas import tpu_sc as plsc

_F = 128
_NC = 2
_NS = 16
_NW = _NC * _NS
_LANES = 16
_CHUNK = 64


@functools.lru_cache(maxsize=None)
def _make_kernel(P: int):
    assert P % _NW == 0
    ppw = P // _NW
    assert ppw % _CHUNK == 0
    nch = ppw // _CHUNK
    assert nch % 2 == 0
    rows = 2 * _CHUNK

    mesh = plsc.VectorSubcoreMesh(
        core_axis_name="c", subcore_axis_name="s",
        num_cores=_NC, num_subcores=_NS)

    nbuf = 4
    assert nch % nbuf == 0

    @functools.partial(
        pl.kernel,
        out_type=jax.ShapeDtypeStruct((2 * P, _F), jnp.float32),
        mesh=mesh,
        scratch_types=[
            pltpu.VMEM((2 * ppw,), jnp.int32),
            [pltpu.VMEM((rows, _F), jnp.float32) for _ in range(nbuf)],
            [pltpu.SemaphoreType.DMA for _ in range(nbuf)],
            [pltpu.SemaphoreType.DMA for _ in range(nbuf)],
        ],
        compiler_params=pltpu.CompilerParams(needs_layout_passes=False),
    )
    def emb(x_hbm, tab_hbm, out_hbm, idx, bufs, gsems, osems):
        wid = lax.axis_index("s") * _NC + lax.axis_index("c")
        base = wid * ppw
        pltpu.sync_copy(x_hbm.at[pl.ds(2 * base, 2 * ppw)], idx)

        def gather(ci, k):
            r0 = ci * rows
            pltpu.async_copy(
                tab_hbm.at[idx.at[pl.ds(r0, rows)]], bufs[k], gsems[k])

        def gather_wait(k):
            pltpu.make_async_copy(
                tab_hbm.at[idx.at[pl.ds(0, rows)]], bufs[k], gsems[k]).wait()

        def writeback(ci, k):
            r0 = 2 * base + ci * rows
            pltpu.async_copy(bufs[k], out_hbm.at[pl.ds(r0, rows)], osems[k])

        def writeback_wait(k):
            pltpu.make_async_copy(
                bufs[k], out_hbm.at[pl.ds(2 * base, rows)], osems[k]).wait()

        def stage(i, c0):
            ci = i * nbuf
            for k in range(nbuf):
                @pl.when(i > 0)
                def _(k=k):
                    writeback_wait(k)
                gather(ci + k, k)
            for k in range(nbuf):
                gather_wait(k)
                writeback(ci + k, k)
            return c0

        lax.fori_loop(0, nch // nbuf, stage, 0)
        for k in range(nbuf):
            writeback_wait(k)

    return emb


def kernel(x, col_embed, row_embed):
    b, n, _ = x.shape
    p = b * n
    tab = jnp.concatenate([col_embed, row_embed], axis=0)
    fused_idx = x.reshape(2 * p) + (jnp.arange(2 * p, dtype=jnp.int32) & 1) * 512
    out = _make_kernel(p)(fused_idx, tab)
    return out.reshape(b, n, 2, _F).swapaxes(2, 3)


# speedup vs baseline: 78.1837x; 1.5360x over previous
import functools

import jax
import jax.numpy as jnp
from jax import lax
from jax.experimental import pallas as pl
from jax.experimental.pallas import tpu as pltpu
from jax.experimental.pallas import tpu_sc as plsc

_F = 128
_NC = 2
_NS = 16
_NW = _NC * _NS
_LANES = 16
_CHUNK = 64


@functools.lru_cache(maxsize=None)
def _make_kernel(P: int):
    assert P % _NW == 0
    ppw = P // _NW
    assert ppw % _CHUNK == 0
    nch = ppw // _CHUNK
    assert nch % 2 == 0
    rows = 2 * _CHUNK

    mesh = plsc.VectorSubcoreMesh(
        core_axis_name="c", subcore_axis_name="s",
        num_cores=_NC, num_subcores=_NS)

    nbuf = 4
    assert nch % nbuf == 0

    @functools.partial(
        pl.kernel,
        out_type=jax.ShapeDtypeStruct((2 * P, _F), jnp.float32),
        mesh=mesh,
        scratch_types=[
            pltpu.VMEM((2 * ppw,), jnp.int32),
            pltpu.VMEM_SHARED((1024, _F), jnp.float32),
            [pltpu.VMEM((rows, _F), jnp.float32) for _ in range(nbuf)],
            [pltpu.SemaphoreType.DMA for _ in range(nbuf)],
            [pltpu.SemaphoreType.DMA for _ in range(nbuf)],
        ],
        compiler_params=pltpu.CompilerParams(needs_layout_passes=False),
    )
    def emb(x_hbm, tab_hbm, out_hbm, idx, stab, bufs, gsems, osems):
        wid = lax.axis_index("s") * _NC + lax.axis_index("c")
        base = wid * ppw
        pltpu.sync_copy(x_hbm.at[pl.ds(2 * base, 2 * ppw)], idx)

        @pl.when(lax.axis_index("s") == 0)
        def _():
            pltpu.sync_copy(tab_hbm, stab)

        plsc.subcore_barrier()

        def gather(ci, k):
            r0 = ci * rows
            pltpu.async_copy(
                stab.at[idx.at[pl.ds(r0, rows)]], bufs[k], gsems[k])

        def gather_wait(k):
            pltpu.make_async_copy(
                stab.at[idx.at[pl.ds(0, rows)]], bufs[k], gsems[k]).wait()

        def writeback(ci, k):
            r0 = 2 * base + ci * rows
            pltpu.async_copy(bufs[k], out_hbm.at[pl.ds(r0, rows)], osems[k])

        def writeback_wait(k):
            pltpu.make_async_copy(
                bufs[k], out_hbm.at[pl.ds(2 * base, rows)], osems[k]).wait()

        def stage(i, c0):
            ci = i * nbuf
            for k in range(nbuf):
                @pl.when(i > 0)
                def _(k=k):
                    writeback_wait(k)
                gather(ci + k, k)
            for k in range(nbuf):
                gather_wait(k)
                writeback(ci + k, k)
            return c0

        lax.fori_loop(0, nch // nbuf, stage, 0)
        for k in range(nbuf):
            writeback_wait(k)

    return emb


def kernel(x, col_embed, row_embed):
    b, n, _ = x.shape
    p = b * n
    tab = jnp.concatenate([col_embed, row_embed], axis=0)
    fused_idx = x.reshape(2 * p) + (jnp.arange(2 * p, dtype=jnp.int32) & 1) * 512
    out = _make_kernel(p)(fused_idx, tab)
    return out.reshape(b, n, 2, _F).swapaxes(2, 3)
